# baseline (speedup 1.0000x reference)
import jax
import jax.numpy as jnp
from jax import lax
from jax.experimental import pallas as pl
from jax.experimental.pallas import tpu as pltpu
from jax.experimental.pallas import tpu_sc as plsc

DIM = 64
BATCH = 16384
L = 16
NC = 2
NS = 16
NW = NC * NS
B_W = BATCH // NW
P_W = B_W // 2
CHUNK = 128


def _tec_body(ent_hbm, rel_hbm, hp_hbm, ho_hbm, rp_hbm, ro_hbm, tp_hbm, to_hbm,
              psum_hbm,
              hpi, hoi, rpi, roi, tpi, toi, hbuf, rbuf, tbuf, pbuf, sem):
    wid = lax.axis_index("s") * NC + lax.axis_index("c")
    base = wid * B_W

    for p in range(2):
        pb = base + p * P_W
        pltpu.sync_copy(hp_hbm.at[pl.ds(pb, P_W)], hpi)
        pltpu.sync_copy(ho_hbm.at[pl.ds(pb, P_W)], hoi)
        pltpu.sync_copy(rp_hbm.at[pl.ds(pb, P_W)], rpi)
        pltpu.sync_copy(ro_hbm.at[pl.ds(pb, P_W)], roi)
        pltpu.sync_copy(tp_hbm.at[pl.ds(pb, P_W)], tpi)
        pltpu.sync_copy(to_hbm.at[pl.ds(pb, P_W)], toi)

        copies = []
        for j in range(P_W // CHUNK):
            sl = pl.ds(j * CHUNK, CHUNK)
            copies.append(pltpu.async_copy(ent_hbm.at[hpi.at[sl]], hbuf.at[sl], sem))
            copies.append(pltpu.async_copy(rel_hbm.at[rpi.at[sl]], rbuf.at[sl], sem))
            copies.append(pltpu.async_copy(ent_hbm.at[tpi.at[sl]], tbuf.at[sl], sem))
        for c in copies:
            c.wait()

        def group(g, carry):
            hov = hoi[pl.ds(g * L, L)]
            rov = roi[pl.ds(g * L, L)]
            tov = toi[pl.ds(g * L, L)]
            for j in range(L):
                i = g * L + j
                ho = hov[j]
                ro = rov[j]
                to = tov[j]
                acc = jnp.zeros((L,), jnp.float32)
                for k in range(DIM // L):
                    o = k * L
                    diff = (hbuf[i, pl.ds(ho + o, L)]
                            + rbuf[i, pl.ds(ro + o, L)]
                            - tbuf[i, pl.ds(to + o, L)])
                    acc = acc + diff * diff
                ii = p * P_W + i
                pbuf[ii >> 3, pl.ds((j & 7) * L, L)] = acc
            return carry

        lax.fori_loop(0, P_W // L, group, 0)

    pltpu.sync_copy(pbuf, psum_hbm.at[pl.ds(wid * (B_W // 8), B_W // 8)])


def _tc_norm_body(p_ref, o_ref):
    p = p_ref[...]
    lane_grp = lax.broadcasted_iota(jnp.int32, (128, 8), 0) // L
    out_grp = lax.broadcasted_iota(jnp.int32, (128, 8), 1)
    sel = (lane_grp == out_grp).astype(jnp.float32)
    o_ref[...] = jnp.sqrt(
        lax.dot_general(p, sel, (((1,), (0,)), ((), ())),
                        precision=lax.Precision.HIGHEST,
                        preferred_element_type=jnp.float32))


def kernel(ent_emb, rel_emb, h, r, t):
    h = h.astype(jnp.int32)
    r = r.astype(jnp.int32)
    t = t.astype(jnp.int32)
    ent2 = ent_emb.reshape(ent_emb.shape[0] // 2, 2 * DIM)
    rel2 = rel_emb.reshape(rel_emb.shape[0] // 2, 2 * DIM)
    hp, ho = h >> 1, (h & 1) << 6
    rp, ro = r >> 1, (r & 1) << 6
    tp, to = t >> 1, (t & 1) << 6
    mesh = plsc.VectorSubcoreMesh(core_axis_name="c", subcore_axis_name="s")
    gather_partials = pl.kernel(
        _tec_body,
        mesh=mesh,
        out_type=jax.ShapeDtypeStruct((BATCH // 8, 8 * L), jnp.float32),
        scratch_types=[
            pltpu.VMEM((P_W,), jnp.int32),
            pltpu.VMEM((P_W,), jnp.int32),
            pltpu.VMEM((P_W,), jnp.int32),
            pltpu.VMEM((P_W,), jnp.int32),
            pltpu.VMEM((P_W,), jnp.int32),
            pltpu.VMEM((P_W,), jnp.int32),
            pltpu.VMEM((P_W, 2 * DIM), jnp.float32),
            pltpu.VMEM((P_W, 2 * DIM), jnp.float32),
            pltpu.VMEM((P_W, 2 * DIM), jnp.float32),
            pltpu.VMEM((B_W // 8, 8 * L), jnp.float32),
            pltpu.SemaphoreType.DMA,
        ],
    )
    psums = gather_partials(ent2, rel2, hp, ho, rp, ro, tp, to)
    norms = pl.pallas_call(
        _tc_norm_body,
        out_shape=jax.ShapeDtypeStruct((BATCH // 8, 8), jnp.float32),
    )(psums)
    return norms.reshape(BATCH)


# speedup vs baseline: 1.6682x; 1.6682x over previous
import jax
import jax.numpy as jnp
from jax import lax
from jax.experimental import pallas as pl
from jax.experimental.pallas import tpu as pltpu
from jax.experimental.pallas import tpu_sc as plsc

DIM = 64
BATCH = 16384
L = 16
NC = 2
NS = 16
NW = NC * NS
B_W = BATCH // NW
P_W = 128
NPASS = B_W // P_W


def _tec_body(ent_hbm, rel_hbm, h_hbm, r_hbm, t_hbm, psum_hbm,
              hidx, ridx, tidx, hbuf, rbuf, tbuf, pbuf, sem):
    wid = lax.axis_index("s") * NC + lax.axis_index("c")
    base = wid * B_W

    def one_pass(p, carry):
        pb = base + p * P_W
        pltpu.sync_copy(h_hbm.at[pl.ds(pb, P_W)], hidx)
        pltpu.sync_copy(r_hbm.at[pl.ds(pb, P_W)], ridx)
        pltpu.sync_copy(t_hbm.at[pl.ds(pb, P_W)], tidx)

        def fire(g, carry2):
            hv = hidx[pl.ds(g * L, L)]
            rv = ridx[pl.ds(g * L, L)]
            tv = tidx[pl.ds(g * L, L)]
            for j in range(L):
                e = g * L + j
                pltpu.async_copy(ent_hbm.at[hv[j]], hbuf.at[e], sem)
                pltpu.async_copy(rel_hbm.at[rv[j]], rbuf.at[e], sem)
                pltpu.async_copy(ent_hbm.at[tv[j]], tbuf.at[e], sem)
            return carry2

        lax.fori_loop(0, P_W // L, fire, 0)

        pltpu.make_async_copy(ent_hbm.at[pl.ds(0, P_W)], hbuf, sem).wait()
        pltpu.make_async_copy(ent_hbm.at[pl.ds(0, P_W)], tbuf, sem).wait()
        pltpu.make_async_copy(rel_hbm.at[pl.ds(0, P_W)], rbuf, sem).wait()

        def group(g, carry2):
            for j in range(L):
                e = g * L + j
                acc = jnp.zeros((L,), jnp.float32)
                for k in range(DIM // L):
                    sl = pl.ds(k * L, L)
                    diff = hbuf[e, sl] + rbuf[e, sl] - tbuf[e, sl]
                    acc = acc + diff * diff
                pbuf[p * (P_W // 8) + (e >> 3), pl.ds((j & 7) * L, L)] = acc
            return carry2

        lax.fori_loop(0, P_W // L, group, 0)
        return carry

    lax.fori_loop(0, NPASS, one_pass, 0)

    pltpu.sync_copy(pbuf, psum_hbm.at[pl.ds(wid * (B_W // 8), B_W // 8)])


def _tc_norm_body(p_ref, o_ref):
    p = p_ref[...]
    lane_grp = lax.broadcasted_iota(jnp.int32, (128, 8), 0) // L
    out_grp = lax.broadcasted_iota(jnp.int32, (128, 8), 1)
    sel = (lane_grp == out_grp).astype(jnp.float32)
    o_ref[...] = jnp.sqrt(
        lax.dot_general(p, sel, (((1,), (0,)), ((), ())),
                        precision=lax.Precision.HIGHEST,
                        preferred_element_type=jnp.float32))


def kernel(ent_emb, rel_emb, h, r, t):
    h = h.astype(jnp.int32)
    r = r.astype(jnp.int32)
    t = t.astype(jnp.int32)
    mesh = plsc.VectorSubcoreMesh(core_axis_name="c", subcore_axis_name="s")
    gather_partials = pl.kernel(
        _tec_body,
        mesh=mesh,
        out_type=jax.ShapeDtypeStruct((BATCH // 8, 8 * L), jnp.float32),
        scratch_types=[
            pltpu.VMEM((P_W,), jnp.int32),
            pltpu.VMEM((P_W,), jnp.int32),
            pltpu.VMEM((P_W,), jnp.int32),
            pltpu.VMEM((P_W, DIM), jnp.float32),
            pltpu.VMEM((P_W, DIM), jnp.float32),
            pltpu.VMEM((P_W, DIM), jnp.float32),
            pltpu.VMEM((B_W // 8, 8 * L), jnp.float32),
            pltpu.SemaphoreType.DMA,
        ],
    )
    psums = gather_partials(ent_emb, rel_emb, h, r, t)
    norms = pl.pallas_call(
        _tc_norm_body,
        out_shape=jax.ShapeDtypeStruct((BATCH // 8, 8), jnp.float32),
    )(psums)
    return norms.reshape(BATCH)
